# baseline (speedup 1.0000x reference)
import functools

import jax
import jax.numpy as jnp
from jax.experimental import pallas as pl


def _matmul_kernel(x_ref, w_ref, o_ref):
    o_ref[...] = jnp.dot(x_ref[...], w_ref[...],
                         preferred_element_type=jnp.float32)


def _matmul(x, w):
    n, k = x.shape
    k2, m = w.shape
    bn = 1000
    return pl.pallas_call(
        _matmul_kernel,
        grid=(n // bn,),
        in_specs=[pl.BlockSpec((bn, k), lambda i: (i, 0)),
                  pl.BlockSpec((k, m), lambda i: (0, 0))],
        out_specs=pl.BlockSpec((bn, m), lambda i: (i, 0)),
        out_shape=jax.ShapeDtypeStruct((n, m), jnp.float32),
    )(x, w)


def _weighted_dimwise_median(msg, w, dst, n_nodes):
    E = msg.shape[0]
    total = jax.ops.segment_sum(w, dst, num_segments=n_nodes)
    offsets = jnp.concatenate([jnp.zeros((1,), total.dtype), jnp.cumsum(total)])[:-1]

    def one_dim(v):
        order = jnp.lexsort((v, dst))
        v_s = v[order]
        w_s = w[order]
        dst_s = dst[order]
        c = jnp.cumsum(w_s)
        within = c - offsets[dst_s]
        mask = within >= 0.5 * total[dst_s]
        pos = jnp.where(mask, jnp.arange(E), E)
        first = jax.ops.segment_min(pos, dst_s, num_segments=n_nodes)
        first = jnp.clip(first, 0, E - 1)
        med = v_s[first]
        return jnp.where(total > 0, med, 0.0)

    med = jax.vmap(one_dim, in_axes=1, out_axes=1)(msg)
    return total[:, None] * med


def kernel(x, edge_index, edge_weight, W1, b1, W2, b2):
    src = edge_index[0]
    dst = edge_index[1]
    n = x.shape[0]
    h = _matmul(x, W1)
    h = _weighted_dimwise_median(h[src], edge_weight, dst, n) + b1
    h = jax.nn.relu(h)
    h = _matmul(h, W2)
    h = _weighted_dimwise_median(h[src], edge_weight, dst, n) + b2
    return h
